# baseline (speedup 1.0000x reference)
import functools

import jax
import jax.numpy as jnp
from jax import lax
from jax.experimental import pallas as pl
from jax.experimental.pallas import tpu as pltpu
from jax.experimental.pallas import tpu_sc as plsc

B = 16384
D = 64
H1 = 128
H2 = 64
EPS_BN = 1e-5
EPS_LN = 1e-5
TEMP = 0.07

NC = 2
NS = 16
NW = NC * NS
B_PER_W = B // NW
CHUNK = 128
NCHUNK = B_PER_W // CHUNK


def _sc_gather_body(ue_tab, ie_tab, uidx, iidx, ue_out, ie_out,
                    uidx_v, iidx_v, urows, irows, sem):
    wid = lax.axis_index("s") * NC + lax.axis_index("c")
    wbase = wid * B_PER_W
    for c in range(NCHUNK):
        base = wbase + c * CHUNK
        pltpu.sync_copy(uidx.at[pl.ds(base, CHUNK)], uidx_v)
        pltpu.sync_copy(iidx.at[pl.ds(base, CHUNK)], iidx_v)
        cu = pltpu.async_copy(ue_tab.at[uidx_v], urows, sem)
        ci = pltpu.async_copy(ie_tab.at[iidx_v], irows, sem)
        cu.wait()
        ci.wait()
        pltpu.sync_copy(urows, ue_out.at[pl.ds(base, CHUNK)])
        pltpu.sync_copy(irows, ie_out.at[pl.ds(base, CHUNK)])


def _sc_gather(ue_tab, ie_tab, uidx, iidx):
    mesh = plsc.VectorSubcoreMesh(core_axis_name="c", subcore_axis_name="s",
                                  num_cores=NC, num_subcores=NS)
    fn = pl.kernel(
        _sc_gather_body,
        out_type=(jax.ShapeDtypeStruct((B, D), jnp.float32),
                  jax.ShapeDtypeStruct((B, D), jnp.float32)),
        mesh=mesh,
        scratch_types=(
            pltpu.VMEM((CHUNK,), jnp.int32),
            pltpu.VMEM((CHUNK,), jnp.int32),
            pltpu.VMEM((CHUNK, D), jnp.float32),
            pltpu.VMEM((CHUNK, D), jnp.float32),
            pltpu.SemaphoreType.DMA,
        ),
        compiler_params=pltpu.CompilerParams(use_tc_tiling_on_sc=False),
    )
    return fn(ue_tab, ie_tab, uidx, iidx)


BS = 1024


def _tc_body(uf_ref, ue_ref, cidx_ref, sidx_ref, if_ref, ie_ref,
             ce_ref, se_ref,
             uW1_ref, ub1_ref, us1_ref, ube1_ref, uW2_ref, ub2_ref, us2_ref,
             ube2_ref, uW3_ref, ub3_ref,
             iW1_ref, ib1_ref, is1_ref, ibe1_ref, iW2_ref, ib2_ref, is2_ref,
             ibe2_ref, iW3_ref, ib3_ref,
             lng_ref, lnb_ref, out_ref):
    f32 = jnp.float32

    cidx = cidx_ref[...]
    sidx = sidx_ref[...]
    lanes = lax.broadcasted_iota(jnp.int32, (BS, 32), 1)
    onec = (cidx[:, None] == lanes).astype(f32)
    ones = (sidx[:, None] == lanes).astype(f32)
    ce = jnp.dot(onec, ce_ref[...], preferred_element_type=f32)
    se = jnp.dot(ones, se_ref[...], preferred_element_type=f32)

    ux = jnp.concatenate([uf_ref[...], ue_ref[...], ce, se], axis=-1)
    h = jnp.maximum(jnp.dot(ux, uW1_ref[...], preferred_element_type=f32)
                    + ub1_ref[...], 0.0)
    h = h * us1_ref[...] + ube1_ref[...]
    h = jnp.maximum(jnp.dot(h, uW2_ref[...], preferred_element_type=f32)
                    + ub2_ref[...], 0.0)
    h = h * us2_ref[...] + ube2_ref[...]
    uo = jnp.dot(h, uW3_ref[...], preferred_element_type=f32) + ub3_ref[...]

    ix = jnp.concatenate([if_ref[...], ie_ref[...]], axis=-1)
    h = jnp.maximum(jnp.dot(ix, iW1_ref[...], preferred_element_type=f32)
                    + ib1_ref[...], 0.0)
    h = h * is1_ref[...] + ibe1_ref[...]
    h = jnp.maximum(jnp.dot(h, iW2_ref[...], preferred_element_type=f32)
                    + ib2_ref[...], 0.0)
    h = h * is2_ref[...] + ibe2_ref[...]
    io = jnp.dot(h, iW3_ref[...], preferred_element_type=f32) + ib3_ref[...]

    def ln(x):
        mu = jnp.mean(x, axis=-1, keepdims=True)
        xc = x - mu
        var = jnp.mean(xc * xc, axis=-1, keepdims=True)
        return xc * lax.rsqrt(var + EPS_LN) * lng_ref[...] + lnb_ref[...]

    uo = ln(uo)
    io = ln(io)
    un = jnp.sum(uo * uo, axis=-1, keepdims=True)
    inn = jnp.sum(io * io, axis=-1, keepdims=True)
    dot = jnp.sum(uo * io, axis=-1, keepdims=True)
    denom = jnp.maximum(jnp.sqrt(un), 1e-12) * jnp.maximum(jnp.sqrt(inn), 1e-12)
    out_ref[...] = dot / denom * (1.0 / TEMP)


def _tc_towers(uf, ue, cidx, sidx, if_, ie, cep, sep,
               uW1t, ub1, us1, ube1, uW2t, ub2, us2, ube2, uW3t, ub3,
               iW1t, ib1, is1, ibe1, iW2t, ib2, is2, ibe2, iW3t, ib3,
               lng, lnb):
    grid = (B // BS,)
    row = lambda i: (i, 0)
    rep2 = lambda i: (0, 0)
    in_specs = [
        pl.BlockSpec((BS, D), row),
        pl.BlockSpec((BS, D), row),
        pl.BlockSpec((BS,), lambda i: (i,)),
        pl.BlockSpec((BS,), lambda i: (i,)),
        pl.BlockSpec((BS, D), row),
        pl.BlockSpec((BS, D), row),
        pl.BlockSpec((32, D), rep2),
        pl.BlockSpec((32, D), rep2),
    ]
    weights = [uW1t, ub1, us1, ube1, uW2t, ub2, us2, ube2, uW3t, ub3,
               iW1t, ib1, is1, ibe1, iW2t, ib2, is2, ibe2, iW3t, ib3,
               lng, lnb]
    for w in weights:
        in_specs.append(pl.BlockSpec(w.shape, rep2))
    return pl.pallas_call(
        _tc_body,
        grid=grid,
        in_specs=in_specs,
        out_specs=pl.BlockSpec((BS, 1), row),
        out_shape=jax.ShapeDtypeStruct((B, 1), jnp.float32),
    )(uf, ue, cidx, sidx, if_, ie, cep, sep, *weights)


def kernel(user_idx, user_features, user_color_idx, user_size_idx, item_idx,
           item_features, ue_tab, ie_tab, ce_tab, se_tab,
           uW1, ub1, ug1, ube1, uW2, ub2, ug2, ube2, uW3, ub3,
           iW1, ib1, ig1, ibe1, iW2, ib2, ig2, ibe2, iW3, ib3,
           ln_g, ln_b):
    f32 = jnp.float32
    uidx = user_idx.astype(jnp.int32)
    iidx = item_idx.astype(jnp.int32)
    cidx = user_color_idx.astype(jnp.int32)
    sidx = user_size_idx.astype(jnp.int32)

    ue, ie = _sc_gather(ue_tab, ie_tab, uidx, iidx)

    bns = 1.0 / jnp.sqrt(jnp.asarray(1.0 + EPS_BN, f32))
    cep = jnp.zeros((32, D), f32).at[:22].set(ce_tab)
    sep = jnp.zeros((32, D), f32).at[:18].set(se_tab)
    row2 = lambda v: v.reshape(1, -1)
    args = (user_features, ue, cidx, sidx, item_features, ie, cep, sep,
            uW1.T, row2(ub1), row2(ug1 * bns), row2(ube1),
            uW2.T, row2(ub2), row2(ug2 * bns), row2(ube2),
            uW3.T, row2(ub3),
            iW1.T, row2(ib1), row2(ig1 * bns), row2(ibe1),
            iW2.T, row2(ib2), row2(ig2 * bns), row2(ibe2),
            iW3.T, row2(ib3),
            row2(ln_g), row2(ln_b))
    return _tc_towers(*args)
